# Initial kernel scaffold; baseline (speedup 1.0000x reference)
import jax
import jax.numpy as jnp
from jax.experimental import pallas as pl


def kernel(data, time, weekday, emb_time, emb_weekday):
    raise NotImplementedError("write your pallas kernel here")


# speedup vs baseline: 1.1620x; 1.1620x over previous
import functools

import jax
import jax.numpy as jnp
from jax import lax
from jax.experimental import pallas as pl
from jax.experimental.pallas import tpu as pltpu
from jax.experimental.pallas import tpu_sc as plsc

B, L = 4096, 50
N = B * L
D_DATA, D_T, D_W = 64, 32, 32
D_OUT = D_DATA + D_T + D_W
NUM_CORES, NUM_SUBCORES = 2, 16
NW = NUM_CORES * NUM_SUBCORES
ROWS_PER_W = N // NW
CHUNK = 128
NCHUNK = ROWS_PER_W // CHUNK

_mesh = plsc.VectorSubcoreMesh(core_axis_name="c", subcore_axis_name="s")


@functools.partial(
    pl.kernel,
    mesh=_mesh,
    compiler_params=pltpu.CompilerParams(use_tc_tiling_on_sc=False),
    out_type=jax.ShapeDtypeStruct((N, D_OUT), jnp.float32),
    scratch_types=[
        pltpu.VMEM((CHUNK,), jnp.int32),
        pltpu.VMEM((CHUNK,), jnp.int32),
        pltpu.VMEM((CHUNK, D_T), jnp.float32),
        pltpu.VMEM((CHUNK, D_W), jnp.float32),
        pltpu.VMEM((CHUNK, D_DATA), jnp.float32),
        pltpu.SemaphoreType.DMA,
        pltpu.SemaphoreType.DMA,
    ],
)
def _embed_sc(data_hbm, time_hbm, wday_hbm, et_hbm, ew_hbm, out_hbm,
              tidx, widx, tbuf, wbuf, dbuf, sem_t, sem_w):
    wid = lax.axis_index("s") * NUM_CORES + lax.axis_index("c")
    base0 = wid * ROWS_PER_W

    def body(j, carry):
        base = base0 + j * CHUNK
        pltpu.sync_copy(time_hbm.at[pl.ds(base, CHUNK)], tidx)
        pltpu.sync_copy(wday_hbm.at[pl.ds(base, CHUNK)], widx)
        ct = pltpu.async_copy(et_hbm.at[tidx], tbuf, sem_t)
        cw = pltpu.async_copy(ew_hbm.at[widx], wbuf, sem_w)
        pltpu.sync_copy(data_hbm.at[pl.ds(base, CHUNK), :], dbuf)
        pltpu.sync_copy(dbuf, out_hbm.at[pl.ds(base, CHUNK), pl.ds(0, D_DATA)])
        ct.wait()
        pltpu.sync_copy(tbuf, out_hbm.at[pl.ds(base, CHUNK), pl.ds(D_DATA, D_T)])
        cw.wait()
        pltpu.sync_copy(wbuf, out_hbm.at[pl.ds(base, CHUNK), pl.ds(D_DATA + D_T, D_W)])
        return carry

    lax.fori_loop(0, NCHUNK, body, 0)


def kernel(data, time, weekday, emb_time, emb_weekday):
    d = data.reshape(N, D_DATA)
    t = time.reshape(N).astype(jnp.int32)
    w = weekday.reshape(N).astype(jnp.int32)
    out = _embed_sc(d, t, w, emb_time, emb_weekday)
    return out.reshape(B, L, D_OUT)
